# baseline (speedup 1.0000x reference)
import dataclasses
import functools

import jax
import jax.numpy as jnp
from jax import lax
from jax.experimental import pallas as pl
from jax.experimental.pallas import tpu as pltpu
from jax.experimental.pallas import tpu_sc as plsc

B = 16384
D = 128
DW = D // 2
P = 1024
A = 512
NC = 2
NS = 16
NW = NC * NS
RPW = B // 2 // NW
RCH = 128


def _tc_prep(pos_embed, action_embed):
    def _pack(x, rows):
        u = lax.bitcast_convert_type(x.astype(jnp.bfloat16), jnp.uint16)
        lo = u[:, :DW].astype(jnp.uint32)
        hi = u[:, DW:].astype(jnp.uint32)
        w = lo | (hi << 16)
        half = rows // 2
        return lax.bitcast_convert_type(
            jnp.concatenate([w[:half], w[half:]], axis=1), jnp.int32)

    def body(pos_ref, aemb_ref, post_ref, aembt_ref):
        post_ref[...] = _pack(pos_ref[...], P)
        aembt_ref[...] = _pack(aemb_ref[...], A)

    return pl.pallas_call(
        body,
        out_shape=[
            jax.ShapeDtypeStruct((P // 2, D), jnp.int32),
            jax.ShapeDtypeStruct((A // 2, D), jnp.int32),
        ],
    )(pos_embed, action_embed)


def _sc_gather_sum(tok_s, act_idx, post, aembt):
    mesh = plsc.VectorSubcoreMesh(core_axis_name="c", subcore_axis_name="s")
    cp = pltpu.CompilerParams()
    if "needs_layout_passes" in pltpu.CompilerParams.__dataclass_fields__:
        cp = dataclasses.replace(cp, needs_layout_passes=False)

    def _row(tab_v, c, half_rows):
        cw = (c % half_rows) * D + (c // half_rows) * DW
        return [
            plsc.bitcast(tab_v[pl.ds(cw + 16 * j, 16)], jnp.bfloat16)
            for j in range(4)
        ]

    @functools.partial(
        pl.kernel,
        out_type=jax.ShapeDtypeStruct((B * DW,), jnp.int32),
        mesh=mesh,
        compiler_params=cp,
        scratch_types=[
            pltpu.VMEM((P * DW,), jnp.int32),
            pltpu.VMEM((A * DW,), jnp.int32),
            pltpu.VMEM((2 * 9 * RCH,), jnp.int32),
            pltpu.VMEM((RCH * D,), jnp.int32),
            pltpu.SemaphoreType.DMA,
        ],
    )
    def k(t0, t1, t2, t3, t4, t5, t6, t7, aidx_hbm, post_hbm, aembt_hbm,
          out_hbm, post_v, aembt_v, idx_v, out_v, sem):
        idx_hbm = [t0, t1, t2, t3, t4, t5, t6, t7, aidx_hbm]
        wid = lax.axis_index("s") * NC + lax.axis_index("c")
        ct0 = pltpu.async_copy(post_hbm, post_v, sem)
        ct1 = pltpu.async_copy(aembt_hbm, aembt_v, sem)
        for chunk in range(RPW // RCH):
            row0 = wid * RPW + chunk * RCH
            cps = [
                pltpu.async_copy(
                    idx_hbm[s].at[pl.ds(h * (B // 2) + row0, RCH)],
                    idx_v.at[pl.ds((h * 9 + s) * RCH, RCH)], sem)
                for h in range(2) for s in range(9)
            ]
            if chunk == 0:
                ct0.wait()
                ct1.wait()
            for c in cps:
                c.wait()

            @pl.loop(0, 2 * RCH, step=16)
            def _(gg):
                h = gg // RCH
                g0 = gg % RCH
                tv = [idx_v[pl.ds((h * 9 + s) * RCH + g0, 16)]
                      for s in range(9)]
                addr = [((tv[s] & (P // 2 - 1)) << 7) + ((tv[s] >> 9) << 6)
                        for s in range(8)]
                addr.append(((tv[8] & (A // 2 - 1)) << 7) + ((tv[8] >> 8) << 6))
                t16 = lax.iota(jnp.int32, 16)
                lt8 = t16 < 8
                tm8 = t16 & 7
                pat0 = jnp.where(lt8, 0, 1)
                out_adj = jnp.where(lt8, 0, D)
                offs = [tm8 + jnp.where(lt8, 8 * q, 8 * ((q + 1) % 8))
                        for q in range(8)]
                offs_out = [offs[q] + out_adj for q in range(8)]
                for p in range(8):
                    pat = pat0 + 2 * p
                    bv = [addr[s].at[pat].get(mode="promise_in_bounds")
                          for s in range(9)]
                    obase = (g0 + 2 * p) * D + h * DW
                    for q in range(8):
                        acc = plsc.bitcast(
                            plsc.load_gather(aembt_v, [bv[8] + offs[q]]),
                            jnp.bfloat16)
                        for s in range(8):
                            acc = acc + plsc.bitcast(
                                plsc.load_gather(post_v, [bv[s] + offs[q]]),
                                jnp.bfloat16)
                        plsc.store_scatter(out_v, [obase + offs_out[q]],
                                           plsc.bitcast(acc, jnp.int32))

            pltpu.sync_copy(out_v, out_hbm.at[pl.ds(row0 * D, RCH * D)])

    return k(*tok_s, act_idx, post, aembt)


def _tc_tail(embp, cont3, W_cont, b_cont, W_out, b_out, gamma, beta):
    BLK = 2048

    def body(embp_ref, cont_ref, wc_ref, bc_ref, wo_ref, bo_ref, g_ref, bt_ref,
             o_ref):
        u = embp_ref[...]
        wc, bc = wc_ref[...], bc_ref[...]

        def half(u64, cont):
            lo = lax.bitcast_convert_type(u64 << 16, jnp.float32)
            hi = lax.bitcast_convert_type(u64 & jnp.int32(-65536), jnp.float32)
            emb = jnp.concatenate([lo, hi], axis=1)
            ce = jnp.dot(cont, wc, preferred_element_type=jnp.float32) + bc
            turn = (emb + ce) * 0.1
            y = jnp.dot(turn, wo_ref[...],
                        preferred_element_type=jnp.float32) + bo_ref[...]
            m = jnp.mean(y, axis=-1, keepdims=True)
            yc = y - m
            v = jnp.mean(yc * yc, axis=-1, keepdims=True)
            return yc * lax.rsqrt(v + 1e-5) * g_ref[...] + bt_ref[...]

        o_ref[0] = half(u[:, :DW], cont_ref[0])
        o_ref[1] = half(u[:, DW:], cont_ref[1])

    full = lambda shape: pl.BlockSpec(shape, lambda i: tuple(0 for _ in shape))
    return pl.pallas_call(
        body,
        grid=(B // 2 // BLK,),
        in_specs=[
            pl.BlockSpec((BLK, D), lambda i: (i, 0)),
            pl.BlockSpec((2, BLK, 9), lambda i: (0, i, 0)),
            full((9, D)),
            full((1, D)),
            full((D, D)),
            full((1, D)),
            full((1, D)),
            full((1, D)),
        ],
        out_specs=pl.BlockSpec((2, BLK, D), lambda i: (0, i, 0)),
        out_shape=jax.ShapeDtypeStruct((2, B // 2, D), jnp.float32),
    )(embp, cont3, W_cont, b_cont, W_out, b_out, gamma, beta)


def kernel(token_positions, continuous, action, pos_embed, action_embed,
           W_cont, b_cont, W_out, b_out, gamma, beta):
    tok = token_positions.astype(jnp.int32)
    act_idx = action.astype(jnp.int32)
    tok_s = [tok[:, s] for s in range(8)]
    post, aembt = _tc_prep(pos_embed, action_embed)
    out_flat = _sc_gather_sum(tok_s, act_idx, post.reshape(P * DW),
                              aembt.reshape(A * DW))
    embp = out_flat.reshape(B // 2, D)
    cont3 = continuous.reshape(2, B // 2, 9)
    out3 = _tc_tail(embp, cont3, W_cont, b_cont.reshape(1, D), W_out,
                    b_out.reshape(1, D), gamma.reshape(1, D), beta.reshape(1, D))
    return out3.reshape(B, D)


# speedup vs baseline: 1.4371x; 1.4371x over previous
import dataclasses
import functools

import jax
import jax.numpy as jnp
from jax import lax
from jax.experimental import pallas as pl
from jax.experimental.pallas import tpu as pltpu
from jax.experimental.pallas import tpu_sc as plsc

B = 16384
D = 128
DW = D // 2
P = 1024
A = 512
NC = 2
NS = 16
NW = NC * NS
RPW = B // 2 // NW
RCH = 128


def _tc_prep(pos_embed, action_embed):
    def _pack(x, rows):
        u = lax.bitcast_convert_type(x.astype(jnp.bfloat16), jnp.uint16)
        lo = u[:, :DW].astype(jnp.uint32)
        hi = u[:, DW:].astype(jnp.uint32)
        w = lo | (hi << 16)
        half = rows // 2
        return lax.bitcast_convert_type(
            jnp.concatenate([w[:half], w[half:]], axis=1), jnp.int32)

    def body(pos_ref, aemb_ref, post_ref, aembt_ref):
        post_ref[...] = _pack(pos_ref[...], P)
        aembt_ref[...] = _pack(aemb_ref[...], A)

    return pl.pallas_call(
        body,
        out_shape=[
            jax.ShapeDtypeStruct((P // 2, D), jnp.int32),
            jax.ShapeDtypeStruct((A // 2, D), jnp.int32),
        ],
    )(pos_embed, action_embed)


def _sc_gather_sum(tok_s, act_idx, post, aembt):
    mesh = plsc.VectorSubcoreMesh(core_axis_name="c", subcore_axis_name="s")
    cp = pltpu.CompilerParams()
    if "needs_layout_passes" in pltpu.CompilerParams.__dataclass_fields__:
        cp = dataclasses.replace(cp, needs_layout_passes=False)

    def _row(tab_v, c, half_rows):
        cw = (c % half_rows) * D + (c // half_rows) * DW
        return [
            plsc.bitcast(tab_v[pl.ds(cw + 16 * j, 16)], jnp.bfloat16)
            for j in range(4)
        ]

    @functools.partial(
        pl.kernel,
        out_type=jax.ShapeDtypeStruct((B * DW,), jnp.int32),
        mesh=mesh,
        compiler_params=cp,
        scratch_types=[
            pltpu.VMEM((P * DW,), jnp.int32),
            pltpu.VMEM((A * DW,), jnp.int32),
            pltpu.VMEM((2 * 9 * RCH,), jnp.int32),
            pltpu.VMEM((RCH * D,), jnp.int32),
            pltpu.SemaphoreType.DMA,
        ],
    )
    def k(t0, t1, t2, t3, t4, t5, t6, t7, aidx_hbm, post_hbm, aembt_hbm,
          out_hbm, post_v, aembt_v, idx_v, out_v, sem):
        idx_hbm = [t0, t1, t2, t3, t4, t5, t6, t7, aidx_hbm]
        wid = lax.axis_index("s") * NC + lax.axis_index("c")
        ct0 = pltpu.async_copy(post_hbm, post_v, sem)
        ct1 = pltpu.async_copy(aembt_hbm, aembt_v, sem)
        for chunk in range(RPW // RCH):
            row0 = wid * RPW + chunk * RCH
            cps = [
                pltpu.async_copy(
                    idx_hbm[s].at[pl.ds(h * (B // 2) + row0, RCH)],
                    idx_v.at[pl.ds((h * 9 + s) * RCH, RCH)], sem)
                for h in range(2) for s in range(9)
            ]
            if chunk == 0:
                ct0.wait()
                ct1.wait()
            for c in cps:
                c.wait()

            @plsc.parallel_loop(0, 2 * RCH, step=16)
            def _(gg):
                h = gg // RCH
                g0 = gg % RCH
                tv = [idx_v[pl.ds((h * 9 + s) * RCH + g0, 16)]
                      for s in range(9)]
                for i in range(16):
                    rl = g0 + i
                    acc = _row(aembt_v, tv[8][i], A // 2)
                    for s in range(8):
                        row = _row(post_v, tv[s][i], P // 2)
                        for j in range(4):
                            acc[j] = acc[j] + row[j]
                    for j in range(4):
                        out_v[pl.ds(rl * D + h * DW + 16 * j, 16)] = (
                            plsc.bitcast(acc[j], jnp.int32))

            pltpu.sync_copy(out_v, out_hbm.at[pl.ds(row0 * D, RCH * D)])

    return k(*tok_s, act_idx, post, aembt)


def _tc_tail(embp, cont3, W_cont, b_cont, W_out, b_out, gamma, beta):
    BLK = 2048

    def body(embp_ref, cont_ref, wc_ref, bc_ref, wo_ref, bo_ref, g_ref, bt_ref,
             o_ref):
        u = embp_ref[...]
        wc, bc = wc_ref[...], bc_ref[...]

        def half(u64, cont):
            lo = lax.bitcast_convert_type(u64 << 16, jnp.float32)
            hi = lax.bitcast_convert_type(u64 & jnp.int32(-65536), jnp.float32)
            emb = jnp.concatenate([lo, hi], axis=1)
            ce = jnp.dot(cont, wc, preferred_element_type=jnp.float32) + bc
            turn = (emb + ce) * 0.1
            y = jnp.dot(turn, wo_ref[...],
                        preferred_element_type=jnp.float32) + bo_ref[...]
            m = jnp.mean(y, axis=-1, keepdims=True)
            yc = y - m
            v = jnp.mean(yc * yc, axis=-1, keepdims=True)
            return yc * lax.rsqrt(v + 1e-5) * g_ref[...] + bt_ref[...]

        o_ref[0] = half(u[:, :DW], cont_ref[0])
        o_ref[1] = half(u[:, DW:], cont_ref[1])

    full = lambda shape: pl.BlockSpec(shape, lambda i: tuple(0 for _ in shape))
    return pl.pallas_call(
        body,
        grid=(B // 2 // BLK,),
        in_specs=[
            pl.BlockSpec((BLK, D), lambda i: (i, 0)),
            pl.BlockSpec((2, BLK, 9), lambda i: (0, i, 0)),
            full((9, D)),
            full((1, D)),
            full((D, D)),
            full((1, D)),
            full((1, D)),
            full((1, D)),
        ],
        out_specs=pl.BlockSpec((2, BLK, D), lambda i: (0, i, 0)),
        out_shape=jax.ShapeDtypeStruct((2, B // 2, D), jnp.float32),
    )(embp, cont3, W_cont, b_cont, W_out, b_out, gamma, beta)


def kernel(token_positions, continuous, action, pos_embed, action_embed,
           W_cont, b_cont, W_out, b_out, gamma, beta):
    tok = token_positions.astype(jnp.int32)
    act_idx = action.astype(jnp.int32)
    tok_s = [tok[:, s] for s in range(8)]
    post, aembt = _tc_prep(pos_embed, action_embed)
    out_flat = _sc_gather_sum(tok_s, act_idx, post.reshape(P * DW),
                              aembt.reshape(A * DW))
    embp = out_flat.reshape(B // 2, D)
    cont3 = continuous.reshape(2, B // 2, 9)
    out3 = _tc_tail(embp, cont3, W_cont, b_cont.reshape(1, D), W_out,
                    b_out.reshape(1, D), gamma.reshape(1, D), beta.reshape(1, D))
    return out3.reshape(B, D)


# speedup vs baseline: 1.5816x; 1.1006x over previous
import dataclasses
import functools

import jax
import jax.numpy as jnp
from jax import lax
from jax.experimental import pallas as pl
from jax.experimental.pallas import tpu as pltpu
from jax.experimental.pallas import tpu_sc as plsc

B = 16384
D = 128
DW = D // 2
P = 1024
A = 512
NC = 2
NS = 16
NW = NC * NS
RPW = B // 2 // NW
RCH = 128


def _tc_prep(pos_embed, action_embed):
    def _pack(x, rows):
        u = lax.bitcast_convert_type(x.astype(jnp.bfloat16), jnp.uint16)
        lo = u[:, :DW].astype(jnp.uint32)
        hi = u[:, DW:].astype(jnp.uint32)
        w = lo | (hi << 16)
        half = rows // 2
        return lax.bitcast_convert_type(
            jnp.concatenate([w[:half], w[half:]], axis=1), jnp.int32)

    def body(pos_ref, aemb_ref, post_ref, aembt_ref):
        post_ref[...] = _pack(pos_ref[...], P)
        aembt_ref[...] = _pack(aemb_ref[...], A)

    return pl.pallas_call(
        body,
        out_shape=[
            jax.ShapeDtypeStruct((P // 2, D), jnp.int32),
            jax.ShapeDtypeStruct((A // 2, D), jnp.int32),
        ],
    )(pos_embed, action_embed)


def _sc_gather_sum(tok_s, act_idx, post, aembt):
    mesh = plsc.VectorSubcoreMesh(core_axis_name="c", subcore_axis_name="s")
    cp = pltpu.CompilerParams()
    if "needs_layout_passes" in pltpu.CompilerParams.__dataclass_fields__:
        cp = dataclasses.replace(cp, needs_layout_passes=False)

    def _row(tab_v, cw):
        return [
            plsc.bitcast(tab_v[pl.ds(cw + 16 * j, 16)], jnp.bfloat16)
            for j in range(4)
        ]

    @functools.partial(
        pl.kernel,
        out_type=jax.ShapeDtypeStruct((B * DW,), jnp.int32),
        mesh=mesh,
        compiler_params=cp,
        scratch_types=[
            pltpu.VMEM((P * DW,), jnp.int32),
            pltpu.VMEM((A * DW,), jnp.int32),
            pltpu.VMEM((2 * 9 * RCH,), jnp.int32),
            pltpu.VMEM((RCH * D,), jnp.int32),
            pltpu.SemaphoreType.DMA,
        ],
    )
    def k(t0, t1, t2, t3, t4, t5, t6, t7, aidx_hbm, post_hbm, aembt_hbm,
          out_hbm, post_v, aembt_v, idx_v, out_v, sem):
        idx_hbm = [t0, t1, t2, t3, t4, t5, t6, t7, aidx_hbm]
        wid = lax.axis_index("s") * NC + lax.axis_index("c")
        ct0 = pltpu.async_copy(post_hbm, post_v, sem)
        ct1 = pltpu.async_copy(aembt_hbm, aembt_v, sem)
        for chunk in range(RPW // RCH):
            row0 = wid * RPW + chunk * RCH
            cps = [
                pltpu.async_copy(
                    idx_hbm[s].at[pl.ds(h * (B // 2) + row0, RCH)],
                    idx_v.at[pl.ds((h * 9 + s) * RCH, RCH)], sem)
                for h in range(2) for s in range(9)
            ]
            if chunk == 0:
                ct0.wait()
                ct1.wait()
            for c in cps:
                c.wait()

            @plsc.parallel_loop(0, 2 * RCH, step=16)
            def _(gg):
                h = gg // RCH
                g0 = gg % RCH
                tv = [idx_v[pl.ds((h * 9 + s) * RCH + g0, 16)]
                      for s in range(9)]
                for i in range(16):
                    rl = g0 + i
                    acc = _row(aembt_v, tv[8][i])
                    for s in range(8):
                        row = _row(post_v, tv[s][i])
                        for j in range(4):
                            acc[j] = acc[j] + row[j]
                    for j in range(4):
                        out_v[pl.ds(rl * D + h * DW + 16 * j, 16)] = (
                            plsc.bitcast(acc[j], jnp.int32))

            pltpu.sync_copy(out_v, out_hbm.at[pl.ds(row0 * D, RCH * D)])

    return k(*tok_s, act_idx, post, aembt)


def _tc_tail(embp, cont3, W_cont, b_cont, W_out, b_out, gamma, beta):
    BLK = 2048

    def body(embp_ref, cont_ref, wc_ref, bc_ref, wo_ref, bo_ref, g_ref, bt_ref,
             o_ref):
        u = embp_ref[...]
        wc, bc = wc_ref[...], bc_ref[...]

        def half(u64, cont):
            lo = lax.bitcast_convert_type(u64 << 16, jnp.float32)
            hi = lax.bitcast_convert_type(u64 & jnp.int32(-65536), jnp.float32)
            emb = jnp.concatenate([lo, hi], axis=1)
            ce = jnp.dot(cont, wc, preferred_element_type=jnp.float32) + bc
            turn = (emb + ce) * 0.1
            y = jnp.dot(turn, wo_ref[...],
                        preferred_element_type=jnp.float32) + bo_ref[...]
            m = jnp.mean(y, axis=-1, keepdims=True)
            yc = y - m
            v = jnp.mean(yc * yc, axis=-1, keepdims=True)
            return yc * lax.rsqrt(v + 1e-5) * g_ref[...] + bt_ref[...]

        o_ref[0] = half(u[:, :DW], cont_ref[0])
        o_ref[1] = half(u[:, DW:], cont_ref[1])

    full = lambda shape: pl.BlockSpec(shape, lambda i: tuple(0 for _ in shape))
    return pl.pallas_call(
        body,
        grid=(B // 2 // BLK,),
        in_specs=[
            pl.BlockSpec((BLK, D), lambda i: (i, 0)),
            pl.BlockSpec((2, BLK, 9), lambda i: (0, i, 0)),
            full((9, D)),
            full((1, D)),
            full((D, D)),
            full((1, D)),
            full((1, D)),
            full((1, D)),
        ],
        out_specs=pl.BlockSpec((2, BLK, D), lambda i: (0, i, 0)),
        out_shape=jax.ShapeDtypeStruct((2, B // 2, D), jnp.float32),
    )(embp, cont3, W_cont, b_cont, W_out, b_out, gamma, beta)


def kernel(token_positions, continuous, action, pos_embed, action_embed,
           W_cont, b_cont, W_out, b_out, gamma, beta):
    tok = token_positions.astype(jnp.int32)
    act_i = action.astype(jnp.int32)
    act_idx = (act_i & (A // 2 - 1)) * D + (act_i >> 8) * DW
    tok_s = [(tok[:, s] & (P // 2 - 1)) * D + (tok[:, s] >> 9) * DW
             for s in range(8)]
    post, aembt = _tc_prep(pos_embed, action_embed)
    out_flat = _sc_gather_sum(tok_s, act_idx, post.reshape(P * DW),
                              aembt.reshape(A * DW))
    embp = out_flat.reshape(B // 2, D)
    cont3 = continuous.reshape(2, B // 2, 9)
    out3 = _tc_tail(embp, cont3, W_cont, b_cont.reshape(1, D), W_out,
                    b_out.reshape(1, D), gamma.reshape(1, D), beta.reshape(1, D))
    return out3.reshape(B, D)
